# baseline (speedup 1.0000x reference)
import functools

import jax
import jax.numpy as jnp
from jax import lax
from jax.experimental import pallas as pl
from jax.experimental.pallas import tpu as pltpu

N_E = 1024
E_DIM = 256
B = 16
S = 1024

_PREC = lax.Precision.DEFAULT


def _vq_body(z_ref, e_ref, zq_ref, idx_ref, loss_ref):
    b = pl.program_id(0)
    z_b = z_ref[0].reshape(E_DIM, S)
    e = e_ref[...]

    en = jnp.sum(e * e, axis=1, keepdims=True)
    prod = lax.dot_general(
        e, z_b, (((1,), (0,)), ((), ())),
        preferred_element_type=jnp.float32, precision=_PREC)
    d = en - 2.0 * prod

    minv = jnp.min(d, axis=0)
    code_iota = lax.broadcasted_iota(jnp.int32, (N_E, S), 0)
    idx = jnp.min(jnp.where(d == minv[None, :], code_iota, N_E), axis=0)

    oh = (code_iota == idx[None, :]).astype(jnp.bfloat16)
    e_hi = e.astype(jnp.bfloat16)
    e_lo = (e - e_hi.astype(jnp.float32)).astype(jnp.bfloat16)
    zq_hi = lax.dot_general(
        e_hi, oh, (((0,), (0,)), ((), ())),
        preferred_element_type=jnp.float32, precision=_PREC)
    zq_lo = lax.dot_general(
        e_lo, oh, (((0,), (0,)), ((), ())),
        preferred_element_type=jnp.float32, precision=_PREC)
    zq = zq_hi + zq_lo

    zq_ref[0] = zq.reshape(E_DIM, 32, 32)
    idx_ref[0, 0] = idx

    zn = jnp.sum(z_b * z_b, axis=0)
    part = jnp.sum(minv + zn)

    @pl.when(b == 0)
    def _():
        loss_ref[0, 0] = 0.0

    loss_ref[0, 0] += part


@jax.jit
def kernel(z, embedding):
    z_q_out, idx_out, loss_acc = pl.pallas_call(
        _vq_body,
        grid=(B,),
        in_specs=[
            pl.BlockSpec((1, E_DIM, 32, 32), lambda b: (b, 0, 0, 0)),
            pl.BlockSpec((N_E, E_DIM), lambda b: (0, 0)),
        ],
        out_specs=[
            pl.BlockSpec((1, E_DIM, 32, 32), lambda b: (b, 0, 0, 0)),
            pl.BlockSpec((1, 1, S), lambda b: (b, 0, 0)),
            pl.BlockSpec(memory_space=pltpu.SMEM, block_shape=(1, 1),
                         index_map=lambda b: (0, 0)),
        ],
        out_shape=[
            jax.ShapeDtypeStruct((B, E_DIM, 32, 32), jnp.float32),
            jax.ShapeDtypeStruct((B, 1, S), jnp.int32),
            jax.ShapeDtypeStruct((1, 1), jnp.float32),
        ],
    )(z, embedding)
    loss = loss_acc[0, 0] / jnp.float32(B * S * E_DIM / 2)
    idx_out = idx_out.reshape(B, 32, 32)
    return (z_q_out, loss, idx_out)


# speedup vs baseline: 2.2645x; 2.2645x over previous
import functools

import jax
import jax.numpy as jnp
from jax import lax
from jax.experimental import pallas as pl
from jax.experimental.pallas import tpu as pltpu

N_E = 1024
E_DIM = 256
B = 16
S = 1024

_PREC = lax.Precision.DEFAULT


def _vq_body(z_ref, e_ref, ehi_ref, elo_ref, zq_ref, idx_ref, loss_ref):
    b = pl.program_id(0)
    z_b = z_ref[0]
    e = e_ref[...]

    en = jnp.sum(e * e, axis=1, keepdims=True)
    prod = lax.dot_general(
        e, z_b, (((1,), (0,)), ((), ())),
        preferred_element_type=jnp.float32, precision=_PREC)
    d = en - 2.0 * prod

    minv = jnp.min(d, axis=0)
    code_iota = lax.broadcasted_iota(jnp.int32, (N_E, S), 0)
    idx = jnp.argmin(d, axis=0).astype(jnp.int32)

    oh = (code_iota == idx[None, :]).astype(jnp.bfloat16)
    e_hi = ehi_ref[...]
    e_lo = elo_ref[...]
    zq_hi = lax.dot_general(
        e_hi, oh, (((0,), (0,)), ((), ())),
        preferred_element_type=jnp.float32, precision=_PREC)
    zq_lo = lax.dot_general(
        e_lo, oh, (((0,), (0,)), ((), ())),
        preferred_element_type=jnp.float32, precision=_PREC)
    zq = zq_hi + zq_lo

    zq_ref[0] = zq
    idx_ref[0, 0] = idx

    zn = jnp.sum(z_b * z_b, axis=0)
    part = jnp.sum(minv + zn)

    @pl.when(b == 0)
    def _():
        loss_ref[0, 0] = 0.0

    loss_ref[0, 0] += part


@jax.jit
def kernel(z, embedding):
    z3 = z.reshape(B, E_DIM, S)
    e_hi = embedding.astype(jnp.bfloat16)
    e_lo = (embedding - e_hi.astype(jnp.float32)).astype(jnp.bfloat16)
    zq3, idx3, loss_acc = pl.pallas_call(
        _vq_body,
        grid=(B,),
        in_specs=[
            pl.BlockSpec((1, E_DIM, S), lambda b: (b, 0, 0)),
            pl.BlockSpec((N_E, E_DIM), lambda b: (0, 0)),
            pl.BlockSpec((N_E, E_DIM), lambda b: (0, 0)),
            pl.BlockSpec((N_E, E_DIM), lambda b: (0, 0)),
        ],
        out_specs=[
            pl.BlockSpec((1, E_DIM, S), lambda b: (b, 0, 0)),
            pl.BlockSpec((1, 1, S), lambda b: (b, 0, 0)),
            pl.BlockSpec(memory_space=pltpu.SMEM, block_shape=(1, 1),
                         index_map=lambda b: (0, 0)),
        ],
        out_shape=[
            jax.ShapeDtypeStruct((B, E_DIM, S), jnp.float32),
            jax.ShapeDtypeStruct((B, 1, S), jnp.int32),
            jax.ShapeDtypeStruct((1, 1), jnp.float32),
        ],
    )(z3, embedding, e_hi, e_lo)
    z_q_out = zq3.reshape(B, E_DIM, 32, 32)
    idx_out = idx3.reshape(B, 32, 32)
    loss = loss_acc[0, 0] / jnp.float32(B * S * E_DIM / 2)
    return (z_q_out, loss, idx_out)
